# baseline (speedup 1.0000x reference)
import jax
import jax.numpy as jnp
from jax import lax
from jax.experimental import pallas as pl
from jax.experimental.pallas import tpu as pltpu
from jax.experimental.pallas import tpu_sc as plsc

N = 10000
N_PAD = 10240
H = 128
HH = H // 2
E = 320000
NC = 2
NS = 16
NT = NC * NS
CHUNK = 80
CH2 = 128
NSLOT = 2
EPT = E // NT
KP = -(-EPT // CH2) // NSLOT * NSLOT + NSLOT
EPT2 = KP * CH2
KD = E // (NT * CHUNK)
TPR = N_PAD // NS
BLK = 400
ZROWS = 128

_MESH = dict(core_axis_name="c", subcore_axis_name="s", num_cores=NC,
             num_subcores=NS)


def _fill1d(ref, n16, val, dtype):
    def body(i, _):
        ref[pl.ds(i * 16, 16)] = jnp.full((16,), val, dtype)
        return 0
    lax.fori_loop(0, n16, body, 0)


def _deg_body(dst_hbm, out_hbm, idx_v, ones_v, zbuf_v, acc_sh):
    cid = lax.axis_index("c")
    sid = lax.axis_index("s")
    wid = cid * NS + sid
    _fill1d(zbuf_v, TPR // 16, 0.0, jnp.float32)
    _fill1d(ones_v, 8, 1.0, jnp.float32)
    pltpu.sync_copy(zbuf_v, acc_sh.at[pl.ds(sid * TPR, TPR)])
    plsc.subcore_barrier()
    pltpu.sync_copy(dst_hbm.at[wid], idx_v)

    def body(j, _):
        pltpu.sync_copy(ones_v.at[pl.ds(0, CHUNK)], acc_sh.at[idx_v.at[j]],
                        add=True)
        return 0
    lax.fori_loop(0, KD, body, 0)
    plsc.subcore_barrier()
    pltpu.sync_copy(acc_sh.at[pl.ds(sid * TPR, TPR)],
                    out_hbm.at[cid, pl.ds(sid * TPR, TPR)])


def _agg_body(src_hbm, dst_hbm, y_hbm, out_hbm, srcv, dstv, r0, r1,
              acc_sh, ga0, ga1, sa0, sa1):
    rows = (r0, r1)
    gsems = (ga0, ga1)
    ssems = (sa0, sa1)
    cid = lax.axis_index("c")
    sid = lax.axis_index("s")
    wid = cid * NS + sid

    def zrow(i, _):
        for k in range(H // 32):
            rows[0][i, pl.ds(k * 32, 32)] = jnp.zeros((32,), jnp.bfloat16)
        return 0
    lax.fori_loop(0, ZROWS, zrow, 0)
    zsrc = rows[0].at[pl.ds(0, ZROWS)]
    for r in range(TPR // ZROWS):
        pltpu.sync_copy(zsrc, acc_sh.at[pl.ds(sid * TPR + r * ZROWS, ZROWS)])

    pltpu.sync_copy(src_hbm.at[wid], srcv)
    pltpu.sync_copy(dst_hbm.at[wid], dstv)
    plsc.subcore_barrier()

    def g_issue(j, p):
        pltpu.async_copy(y_hbm.at[srcv.at[pl.ds(j * CH2, CH2)]],
                         rows[p], gsems[p])

    def g_wait(j, p):
        pltpu.make_async_copy(y_hbm.at[srcv.at[pl.ds(j * CH2, CH2)]],
                              rows[p], gsems[p]).wait()

    def s_issue(j, p):
        pltpu.async_copy(rows[p], acc_sh.at[dstv.at[j]], ssems[p], add=True)

    def s_wait(j, p):
        pltpu.make_async_copy(rows[p], acc_sh.at[dstv.at[j]],
                              ssems[p]).wait()

    for p in range(NSLOT - 1):
        g_issue(p, p)
    T = KP // NSLOT

    def body(t, _):
        for p in range(NSLOT):
            j = NSLOT * t + p
            g_wait(j, p)
            s_issue(j, p)
            pprev = (p + NSLOT - 1) % NSLOT

            @pl.when(j > 0)
            def _():
                s_wait(j - 1, pprev)

            @pl.when(j + NSLOT - 1 < KP)
            def _():
                g_issue(j + NSLOT - 1, pprev)
        return 0
    lax.fori_loop(0, T, body, 0)
    s_wait(KP - 1, NSLOT - 1)
    plsc.subcore_barrier()
    for r in range(TPR // ZROWS):
        s = sid * TPR + r * ZROWS
        pltpu.sync_copy(acc_sh.at[pl.ds(s, ZROWS)],
                        out_hbm.at[cid, pl.ds(s, ZROWS)])


def _dense_body(dp_ref, x_ref, w_ref, ws_ref, b_ref, bs_ref, y_ref, base_ref):
    deg = dp_ref[:, 0:1] + dp_ref[:, 1:2] + 2.0
    dinv = lax.rsqrt(deg)
    xb = x_ref[...]
    xw = jnp.dot(xb, w_ref[...], preferred_element_type=jnp.float32)
    sk = jnp.dot(xb, ws_ref[...], preferred_element_type=jnp.float32)
    y_ref[...] = (dinv * xw).astype(jnp.bfloat16)
    base_ref[...] = sk + b_ref[...] + bs_ref[...] + (2.0 * dinv * dinv) * xw


def _final_body(dp_ref, acc_ref, base_ref, o_ref):
    deg = dp_ref[:, 0:1] + dp_ref[:, 1:2] + 2.0
    dinv = lax.rsqrt(deg)
    agg = (acc_ref[0].astype(jnp.float32) +
           acc_ref[1].astype(jnp.float32))
    o = dinv * agg + base_ref[...]
    o_ref[...] = jnp.where(o > 0, o, 0.1 * (jnp.exp(o) - 1.0))


def _deg_call(dst):
    return pl.kernel(
        _deg_body,
        out_type=jax.ShapeDtypeStruct((NC, N_PAD), jnp.float32),
        mesh=plsc.VectorSubcoreMesh(**_MESH),
        scratch_types=[
            pltpu.VMEM((KD, CHUNK), jnp.int32),
            pltpu.VMEM((128,), jnp.float32),
            pltpu.VMEM((TPR,), jnp.float32),
            pltpu.VMEM_SHARED((N_PAD,), jnp.float32),
        ],
    )(dst)


def _agg_call(src, dst, ycat):
    return pl.kernel(
        _agg_body,
        out_type=jax.ShapeDtypeStruct((NC, N_PAD, H), jnp.bfloat16),
        mesh=plsc.VectorSubcoreMesh(**_MESH),
        compiler_params=pltpu.CompilerParams(use_tc_tiling_on_sc=False),
        scratch_types=[
            pltpu.VMEM((EPT2,), jnp.int32),
            pltpu.VMEM((KP, CH2), jnp.int32),
            pltpu.VMEM((CH2, H), jnp.bfloat16),
            pltpu.VMEM((CH2, H), jnp.bfloat16),
            pltpu.VMEM_SHARED((N_PAD, H), jnp.bfloat16),
            pltpu.SemaphoreType.DMA,
            pltpu.SemaphoreType.DMA,
            pltpu.SemaphoreType.DMA,
            pltpu.SemaphoreType.DMA,
        ],
    )(src, dst, ycat)


def _dense_call(deg_t, x, W, W_skip, b, bs):
    grid = N // BLK
    return pl.pallas_call(
        _dense_body,
        grid=(grid,),
        in_specs=[
            pl.BlockSpec((BLK, NC), lambda i: (i, 0)),
            pl.BlockSpec((BLK, H), lambda i: (i, 0)),
            pl.BlockSpec((H, H), lambda i: (0, 0)),
            pl.BlockSpec((H, H), lambda i: (0, 0)),
            pl.BlockSpec((1, H), lambda i: (0, 0)),
            pl.BlockSpec((1, H), lambda i: (0, 0)),
        ],
        out_specs=[
            pl.BlockSpec((BLK, H), lambda i: (i, 0)),
            pl.BlockSpec((BLK, H), lambda i: (i, 0)),
        ],
        out_shape=[
            jax.ShapeDtypeStruct((N_PAD, H), jnp.bfloat16),
            jax.ShapeDtypeStruct((N_PAD, H), jnp.float32),
        ],
    )(deg_t, x, W, W_skip, b, bs)


def _final_call(deg_t, acc, base):
    grid = N // BLK
    return pl.pallas_call(
        _final_body,
        grid=(grid,),
        in_specs=[
            pl.BlockSpec((BLK, NC), lambda i: (i, 0)),
            pl.BlockSpec((NC, BLK, H), lambda i: (0, i, 0)),
            pl.BlockSpec((BLK, H), lambda i: (i, 0)),
        ],
        out_specs=pl.BlockSpec((BLK, H), lambda i: (i, 0)),
        out_shape=jax.ShapeDtypeStruct((N, H), jnp.float32),
    )(deg_t, acc, base)


def kernel(x, edge_index, W, b, W_skip, b_skip):
    pad_dst = jnp.broadcast_to(N + jnp.arange(EPT2 - EPT, dtype=jnp.int32)
                               % (N_PAD - N), (NT, EPT2 - EPT))
    src = jnp.pad(edge_index[0].reshape(NT, EPT), ((0, 0), (0, EPT2 - EPT)),
                  constant_values=0)
    dst = jnp.concatenate(
        [edge_index[1].reshape(NT, EPT), pad_dst], axis=1).reshape(NT, KP, CH2)
    dst_d = edge_index[1].reshape(NT, KD, CHUNK)
    deg_parts = _deg_call(dst_d)
    deg_t = deg_parts.T
    y3, base = _dense_call(deg_t, x, W, W_skip, b.reshape(1, H),
                           b_skip.reshape(1, H))
    acc = _agg_call(src, dst, y3)
    return _final_call(deg_t, acc, base)


# speedup vs baseline: 1.2764x; 1.2764x over previous
import jax
import jax.numpy as jnp
from jax import lax
from jax.experimental import pallas as pl
from jax.experimental.pallas import tpu as pltpu
from jax.experimental.pallas import tpu_sc as plsc

N = 10000
N_PAD = 10240
H = 128
HH = H // 2
E = 320000
NC = 2
NS = 16
NT = NC * NS
CH2 = 128
NSLOT = 2
EPT = E // NS
KP = -(-EPT // CH2) // NSLOT * NSLOT + NSLOT
EPT2 = KP * CH2
TPR = N_PAD // NS
BLK = 400
ZROWS = 128

_MESH = dict(core_axis_name="c", subcore_axis_name="s", num_cores=NC,
             num_subcores=NS)


def _fill1d(ref, n16, val, dtype):
    def body(i, _):
        ref[pl.ds(i * 16, 16)] = jnp.full((16,), val, dtype)
        return 0
    lax.fori_loop(0, n16, body, 0)


def _deg_body(dst_hbm, out_hbm, idx_v, ones_v, zbuf_v, acc_sh, sem):
    cid = lax.axis_index("c")
    sid = lax.axis_index("s")
    wid = cid * NS + sid
    _fill1d(zbuf_v, TPR // 16, 0.0, jnp.float32)
    _fill1d(ones_v, CH2 // 16, 1.0, jnp.float32)
    pltpu.sync_copy(zbuf_v, acc_sh.at[pl.ds(sid * TPR, TPR)])
    plsc.subcore_barrier()
    pltpu.sync_copy(dst_hbm.at[sid, pl.ds(cid * (KP // 2), KP // 2)], idx_v)

    def fire(j, _):
        pltpu.async_copy(ones_v, acc_sh.at[idx_v.at[j]], sem, add=True)
        return 0
    lax.fori_loop(0, KP // 2, fire, 0)

    def drain(j, _):
        pltpu.make_async_copy(ones_v, acc_sh.at[idx_v.at[j]], sem).wait()
        return 0
    lax.fori_loop(0, KP // 2, drain, 0)
    plsc.subcore_barrier()
    pltpu.sync_copy(acc_sh.at[pl.ds(sid * TPR, TPR)],
                    out_hbm.at[cid, pl.ds(sid * TPR, TPR)])


def _agg_body(src_hbm, dst_hbm, ycat_hbm, out_hbm, srcv, dstv, r0, r1,
              acc_sh, ga0, ga1, sa0, sa1):
    rows = (r0, r1)
    gsems = (ga0, ga1)
    ssems = (sa0, sa1)
    cid = lax.axis_index("c")
    sid = lax.axis_index("s")

    def zrow(i, _):
        for k in range(HH // 32):
            rows[0][i, pl.ds(k * 32, 32)] = jnp.zeros((32,), jnp.bfloat16)
        return 0
    lax.fori_loop(0, ZROWS, zrow, 0)
    zsrc = rows[0].at[pl.ds(0, ZROWS)]
    for r in range(TPR // ZROWS):
        pltpu.sync_copy(zsrc, acc_sh.at[pl.ds(sid * TPR + r * ZROWS, ZROWS)])

    pltpu.sync_copy(src_hbm.at[sid], srcv)
    pltpu.sync_copy(dst_hbm.at[sid], dstv)
    off = cid * N_PAD

    def obody(i, _):
        sl = pl.ds(i * 16, 16)
        srcv[sl] = srcv[sl] + off
        return 0
    lax.fori_loop(0, EPT2 // 16, obody, 0)
    plsc.subcore_barrier()

    def g_issue(j, p):
        pltpu.async_copy(ycat_hbm.at[srcv.at[pl.ds(j * CH2, CH2)]],
                         rows[p], gsems[p])

    def g_wait(j, p):
        pltpu.make_async_copy(ycat_hbm.at[srcv.at[pl.ds(j * CH2, CH2)]],
                              rows[p], gsems[p]).wait()

    def s_issue(j, p):
        pltpu.async_copy(rows[p], acc_sh.at[dstv.at[j]], ssems[p], add=True)

    def s_wait(j, p):
        pltpu.make_async_copy(rows[p], acc_sh.at[dstv.at[j]],
                              ssems[p]).wait()

    for p in range(NSLOT - 1):
        g_issue(p, p)
    T = KP // NSLOT

    def body(t, _):
        for p in range(NSLOT):
            j = NSLOT * t + p
            g_wait(j, p)
            s_issue(j, p)
            pprev = (p + NSLOT - 1) % NSLOT

            @pl.when(j > 0)
            def _():
                s_wait(j - 1, pprev)

            @pl.when(j + NSLOT - 1 < KP)
            def _():
                g_issue(j + NSLOT - 1, pprev)
        return 0
    lax.fori_loop(0, T, body, 0)
    s_wait(KP - 1, NSLOT - 1)
    plsc.subcore_barrier()
    for r in range(TPR // ZROWS):
        s = sid * TPR + r * ZROWS
        pltpu.sync_copy(acc_sh.at[pl.ds(s, ZROWS)],
                        out_hbm.at[cid, pl.ds(s, ZROWS)])


def _dense_body(dp_ref, x_ref, w_ref, ws_ref, b_ref, bs_ref, y_ref, base_ref):
    deg = dp_ref[:, 0:1] + dp_ref[:, 1:2] + 2.0
    dinv = lax.rsqrt(deg)
    xb = x_ref[...]
    xw = jnp.dot(xb, w_ref[...], preferred_element_type=jnp.float32)
    sk = jnp.dot(xb, ws_ref[...], preferred_element_type=jnp.float32)
    y = (dinv * xw).astype(jnp.bfloat16)
    y_ref[0] = y[:, :HH]
    y_ref[1] = y[:, HH:]
    base_ref[...] = sk + b_ref[...] + bs_ref[...] + (2.0 * dinv * dinv) * xw


def _final_body(dp_ref, acc_ref, base_ref, o_ref):
    deg = dp_ref[:, 0:1] + dp_ref[:, 1:2] + 2.0
    dinv = lax.rsqrt(deg)
    agg = jnp.concatenate([acc_ref[0], acc_ref[1]],
                          axis=1).astype(jnp.float32)
    o = dinv * agg + base_ref[...]
    o_ref[...] = jnp.where(o > 0, o, 0.1 * (jnp.exp(o) - 1.0))


def _deg_call(dst):
    return pl.kernel(
        _deg_body,
        out_type=jax.ShapeDtypeStruct((NC, N_PAD), jnp.float32),
        mesh=plsc.VectorSubcoreMesh(**_MESH),
        compiler_params=pltpu.CompilerParams(use_tc_tiling_on_sc=False),
        scratch_types=[
            pltpu.VMEM((KP // 2, CH2), jnp.int32),
            pltpu.VMEM((CH2,), jnp.float32),
            pltpu.VMEM((TPR,), jnp.float32),
            pltpu.VMEM_SHARED((N_PAD,), jnp.float32),
            pltpu.SemaphoreType.DMA,
        ],
    )(dst)


def _agg_call(src, dst, ycat):
    return pl.kernel(
        _agg_body,
        out_type=jax.ShapeDtypeStruct((NC, N_PAD, HH), jnp.bfloat16),
        mesh=plsc.VectorSubcoreMesh(**_MESH),
        compiler_params=pltpu.CompilerParams(use_tc_tiling_on_sc=False),
        scratch_types=[
            pltpu.VMEM((EPT2,), jnp.int32),
            pltpu.VMEM((KP, CH2), jnp.int32),
            pltpu.VMEM((CH2, HH), jnp.bfloat16),
            pltpu.VMEM((CH2, HH), jnp.bfloat16),
            pltpu.VMEM_SHARED((N_PAD, HH), jnp.bfloat16),
            pltpu.SemaphoreType.DMA,
            pltpu.SemaphoreType.DMA,
            pltpu.SemaphoreType.DMA,
            pltpu.SemaphoreType.DMA,
        ],
    )(src, dst, ycat)


def _dense_call(deg_t, x, W, W_skip, b, bs):
    grid = N // BLK
    return pl.pallas_call(
        _dense_body,
        grid=(grid,),
        in_specs=[
            pl.BlockSpec((BLK, NC), lambda i: (i, 0)),
            pl.BlockSpec((BLK, H), lambda i: (i, 0)),
            pl.BlockSpec((H, H), lambda i: (0, 0)),
            pl.BlockSpec((H, H), lambda i: (0, 0)),
            pl.BlockSpec((1, H), lambda i: (0, 0)),
            pl.BlockSpec((1, H), lambda i: (0, 0)),
        ],
        out_specs=[
            pl.BlockSpec((NC, BLK, HH), lambda i: (0, i, 0)),
            pl.BlockSpec((BLK, H), lambda i: (i, 0)),
        ],
        out_shape=[
            jax.ShapeDtypeStruct((NC, N_PAD, HH), jnp.bfloat16),
            jax.ShapeDtypeStruct((N_PAD, H), jnp.float32),
        ],
    )(deg_t, x, W, W_skip, b, bs)


def _final_call(deg_t, acc, base):
    grid = N // BLK
    return pl.pallas_call(
        _final_body,
        grid=(grid,),
        in_specs=[
            pl.BlockSpec((BLK, NC), lambda i: (i, 0)),
            pl.BlockSpec((NC, BLK, HH), lambda i: (0, i, 0)),
            pl.BlockSpec((BLK, H), lambda i: (i, 0)),
        ],
        out_specs=pl.BlockSpec((BLK, H), lambda i: (i, 0)),
        out_shape=jax.ShapeDtypeStruct((N, H), jnp.float32),
    )(deg_t, acc, base)


def kernel(x, edge_index, W, b, W_skip, b_skip):
    pad_dst = jnp.broadcast_to(N + jnp.arange(EPT2 - EPT, dtype=jnp.int32)
                               % (N_PAD - N), (NS, EPT2 - EPT))
    src = jnp.pad(edge_index[0].reshape(NS, EPT), ((0, 0), (0, EPT2 - EPT)),
                  constant_values=0)
    dst = jnp.concatenate(
        [edge_index[1].reshape(NS, EPT), pad_dst], axis=1).reshape(NS, KP, CH2)
    deg_parts = _deg_call(dst)
    deg_t = deg_parts.T
    y3, base = _dense_call(deg_t, x, W, W_skip, b.reshape(1, H),
                           b_skip.reshape(1, H))
    ycat = y3.reshape(NC * N_PAD, HH)
    acc = _agg_call(src, dst, ycat)
    return _final_call(deg_t, acc, base)
